# Initial kernel scaffold; baseline (speedup 1.0000x reference)
import jax
import jax.numpy as jnp
from jax.experimental import pallas as pl


def kernel(map_designs, start_maps, goal_maps):
    raise NotImplementedError("write your pallas kernel here")


# speedup vs baseline: 60.1265x; 60.1265x over previous
import functools

import jax
import jax.numpy as jnp
import numpy as np
from jax import lax
from jax.experimental import pallas as pl
from jax.experimental.pallas import tpu as pltpu
from jax.experimental.pallas import tpu_sc as plsc

L = 16
NC = 2
NS = 16
OPENTH = np.float32(1e36)
CLOSEDV = np.float32(1e37)
INF = np.float32(np.inf)
HALF = np.float32(0.5)


def _astar_sc_kernel(B, N, W, T):
    NCH = N // L
    mesh = plsc.VectorSubcoreMesh(
        core_axis_name="c", subcore_axis_name="s", num_cores=NC, num_subcores=NS
    )

    @functools.partial(
        pl.kernel,
        mesh=mesh,
        compiler_params=pltpu.CompilerParams(needs_layout_passes=False),
        out_type=[
            jax.ShapeDtypeStruct((B, N), jnp.float32),
            jax.ShapeDtypeStruct((B, N), jnp.float32),
        ],
        scratch_types=[
            pltpu.VMEM((N,), jnp.float32),
            pltpu.VMEM((N,), jnp.float32),
            pltpu.VMEM((N,), jnp.float32),
            pltpu.VMEM((N,), jnp.float32),
            pltpu.VMEM((N,), jnp.int32),
            pltpu.VMEM((N,), jnp.float32),
            pltpu.VMEM((L,), jnp.float32),
        ],
    )
    def k(h_hbm, fkey0_hbm, parents0_hbm, path0_hbm, zeros_hbm, goal_hbm,
          hist_out, path_out,
          fkey_v, g_v, h_v, hist_v, parents_v, path_v, goal_v):
        wid = lax.axis_index("s") * NC + lax.axis_index("c")

        @pl.when(wid < B)
        def _():
            pltpu.sync_copy(h_hbm.at[wid], h_v)
            pltpu.sync_copy(fkey0_hbm.at[wid], fkey_v)
            pltpu.sync_copy(parents0_hbm.at[wid], parents_v)
            pltpu.sync_copy(path0_hbm.at[wid], path_v)
            pltpu.sync_copy(zeros_hbm, g_v)
            pltpu.sync_copy(goal_hbm.at[wid], goal_v)

            lane = lax.iota(jnp.int32, L)
            lane_f = lane.astype(jnp.float32)
            goal_f = goal_v[...]
            goal_i = goal_f.astype(jnp.int32)
            t = lane + (lane >= 4).astype(jnp.int32)
            is_nb = lane < 8
            off_y = jnp.where(is_nb, t // 3 - 1, 0)
            off_x = jnp.where(is_nb, t % 3 - 1, 0)
            ones_f = jnp.full((L,), 1.0, jnp.float32)
            inf_v = jnp.full((L,), INF, jnp.float32)
            lane8 = lane == 8
            lane0 = lane == 0
            eights = jnp.full((L,), 8, jnp.int32)

            def splat_min(x):
                for kk in (1, 2, 4, 8):
                    x = jnp.minimum(
                        x, x.at[lane ^ kk].get(mode="promise_in_bounds"))
                return x

            def step(_, solved):
                bv = [inf_v, inf_v, inf_v, inf_v]
                bc = [lane_f * 0.0] * 4
                for c in range(NCH):
                    v = fkey_v[pl.ds(c * L, L)]
                    a = c & 3
                    lt = v < bv[a]
                    bv[a] = jnp.where(lt, v, bv[a])
                    bc[a] = jnp.where(lt, np.float32(c), bc[a])

                def mrg(p, q):
                    v0, c0 = p
                    v1, c1 = q
                    take1 = (v1 < v0) | ((v1 == v0) & (c1 < c0))
                    return (jnp.where(take1, v1, v0),
                            jnp.where(take1, c1, c0))

                v, cid = mrg(mrg((bv[0], bc[0]), (bv[1], bc[1])),
                             mrg((bv[2], bc[2]), (bv[3], bc[3])))
                m = splat_min(v)
                gi = jnp.where(v == m, cid * np.float32(L) + lane_f,
                               np.float32(2 * N))
                n_f = splat_min(gi)
                n_f = jnp.where(m > OPENTH, np.float32(0.0), n_f)
                n_i = n_f.astype(jnp.int32)

                ny = n_i // W
                nx = n_i - ny * W
                my = ny + off_y
                mx = nx + off_x
                inb = is_nb & (my >= 0) & (my < W) & (mx >= 0) & (mx < W)
                mi = jnp.where(inb, my * W + mx, n_i)

                gm = plsc.load_gather(g_v, [mi])
                hm = plsc.load_gather(h_v, [mi])
                fm = plsc.load_gather(fkey_v, [mi])

                hn8 = hm.at[eights].get(mode="promise_in_bounds")
                gval = (gm.at[eights].get(mode="promise_in_bounds")
                        + jnp.where(hn8 > 0.0, np.float32(1.0),
                                    np.float32(0.0)))

                openm = fm < OPENTH
                never = fm == INF
                cond = inb & (hm > 0.0) & (
                    (openm & (gm > gval))
                    | (jnp.logical_not(openm) & never))

                newf = HALF * gval + HALF * jnp.abs(hm)
                close8 = lane8 & (n_f != goal_f)

                plsc.store_scatter(g_v, [mi], gval, mask=cond)
                plsc.store_scatter(parents_v, [mi], n_i, mask=cond)
                plsc.store_scatter(
                    fkey_v, [mi],
                    jnp.where(close8, CLOSEDV, newf),
                    mask=cond | close8)
                return jnp.maximum(
                    solved, jnp.where(n_f == goal_f, 1.0, 0.0))

            solved = lax.fori_loop(0, T, step, jnp.zeros((L,), jnp.float32))

            for c in range(NCH):
                sl = pl.ds(c * L, L)
                hist_v[sl] = jnp.where(fkey_v[sl] == CLOSEDV, 1.0, 0.0)
            plsc.store_scatter(hist_v, [goal_i], solved, mask=lane0)

            loc0 = plsc.load_gather(parents_v, [goal_i])

            def bt(_, loc):
                plsc.store_scatter(path_v, [loc], ones_f, mask=lane0)
                return plsc.load_gather(parents_v, [loc])

            lax.fori_loop(0, T, bt, loc0)

            pltpu.sync_copy(hist_v, hist_out.at[wid])
            pltpu.sync_copy(path_v, path_out.at[wid])

    return k


def kernel(map_designs, start_maps, goal_maps):
    B, H, W = map_designs.shape
    N = H * W
    T = N // 2
    f32 = jnp.float32

    goal_idx = jnp.argmax(goal_maps.reshape(B, N), axis=1).astype(jnp.int32)
    start_idx = jnp.argmax(start_maps.reshape(B, N), axis=1).astype(jnp.int32)

    gy = (goal_idx // W).astype(f32)
    gx = (goal_idx % W).astype(f32)
    dy = jnp.abs(jnp.arange(H, dtype=f32)[None, :, None] - gy[:, None, None])
    dx = jnp.abs(jnp.arange(W, dtype=f32)[None, None, :] - gx[:, None, None])
    dy = jnp.broadcast_to(dy, (B, H, W))
    dx = jnp.broadcast_to(dx, (B, H, W))
    h_cheb = (dy + dx) - jnp.minimum(dy, dx)
    euc = jnp.sqrt(dy ** 2 + dx ** 2)
    h = (h_cheb + 0.001 * euc + map_designs).reshape(B, N).astype(f32)
    h_pack = jnp.where(map_designs.reshape(B, N) == 1.0, h, -h)

    b_ar = jnp.arange(B)
    fkey0 = jnp.full((B, N), jnp.inf, f32)
    fkey0 = fkey0.at[b_ar, start_idx].set(0.5 * h[b_ar, start_idx])
    parents0 = jnp.broadcast_to(goal_idx[:, None], (B, N))
    path0 = goal_maps.reshape(B, N).astype(f32)
    zeros_n = jnp.zeros((N,), f32)
    goal_splat = jnp.broadcast_to(goal_idx[:, None].astype(f32), (B, L))

    hist, path = _astar_sc_kernel(B, N, W, T)(
        h_pack, fkey0, parents0, path0, zeros_n, goal_splat)
    return hist.reshape(B, H, W), path.reshape(B, H, W)


# speedup vs baseline: 105.5983x; 1.7563x over previous
import functools

import jax
import jax.numpy as jnp
import numpy as np
from jax import lax
from jax.experimental import pallas as pl
from jax.experimental.pallas import tpu as pltpu
from jax.experimental.pallas import tpu_sc as plsc

L = 16
NC = 2
NS = 16
OPENTH = np.float32(1e36)
CLOSEDV = np.float32(1e37)
INF = np.float32(np.inf)
HALF = np.float32(0.5)


def _astar_sc_kernel(B, N, W, T):
    NCH = N // L
    mesh = plsc.VectorSubcoreMesh(
        core_axis_name="c", subcore_axis_name="s", num_cores=NC, num_subcores=NS
    )

    @functools.partial(
        pl.kernel,
        mesh=mesh,
        compiler_params=pltpu.CompilerParams(needs_layout_passes=False),
        out_type=[
            jax.ShapeDtypeStruct((B, N), jnp.float32),
            jax.ShapeDtypeStruct((B, N), jnp.float32),
        ],
        scratch_types=[
            pltpu.VMEM((N,), jnp.float32),
            pltpu.VMEM((N,), jnp.float32),
            pltpu.VMEM((N,), jnp.float32),
            pltpu.VMEM((N,), jnp.float32),
            pltpu.VMEM((N,), jnp.int32),
            pltpu.VMEM((N,), jnp.float32),
            pltpu.VMEM((L,), jnp.float32),
        ],
    )
    def k(h_hbm, fkey0_hbm, parents0_hbm, path0_hbm, zeros_hbm, goal_hbm,
          hist_out, path_out,
          fkey_v, g_v, h_v, hist_v, parents_v, path_v, goal_v):
        wid = lax.axis_index("s") * NC + lax.axis_index("c")

        @pl.when(wid < B)
        def _():
            pltpu.sync_copy(h_hbm.at[wid], h_v)
            pltpu.sync_copy(fkey0_hbm.at[wid], fkey_v)
            pltpu.sync_copy(parents0_hbm.at[wid], parents_v)
            pltpu.sync_copy(path0_hbm.at[wid], path_v)
            pltpu.sync_copy(zeros_hbm, g_v)
            pltpu.sync_copy(goal_hbm.at[wid], goal_v)

            lane = lax.iota(jnp.int32, L)
            lane_f = lane.astype(jnp.float32)
            goal_f = goal_v[...]
            goal_i = goal_f.astype(jnp.int32)
            t = lane + (lane >= 4).astype(jnp.int32)
            is_nb = lane < 8
            off_y = jnp.where(is_nb, t // 3 - 1, 0)
            off_x = jnp.where(is_nb, t % 3 - 1, 0)
            ones_f = jnp.full((L,), 1.0, jnp.float32)
            inf_v = jnp.full((L,), INF, jnp.float32)
            lane8 = lane == 8
            lane0 = lane == 0
            eights = jnp.full((L,), 8, jnp.int32)

            def splat_min(x):
                for kk in (1, 2, 4, 8):
                    x = jnp.minimum(
                        x, x.at[lane ^ kk].get(mode="promise_in_bounds"))
                return x

            def step_cond(carry):
                t, solved = carry
                return (t < T) & jnp.logical_not(jnp.any(solved))

            def step(carry):
                t, solved = carry
                bv = [inf_v, inf_v, inf_v, inf_v]
                bc = [lane_f * 0.0] * 4
                for c in range(NCH):
                    v = fkey_v[pl.ds(c * L, L)]
                    a = c & 3
                    lt = v < bv[a]
                    bv[a] = jnp.where(lt, v, bv[a])
                    bc[a] = jnp.where(lt, np.float32(c), bc[a])

                def mrg(p, q):
                    v0, c0 = p
                    v1, c1 = q
                    take1 = (v1 < v0) | ((v1 == v0) & (c1 < c0))
                    return (jnp.where(take1, v1, v0),
                            jnp.where(take1, c1, c0))

                v, cid = mrg(mrg((bv[0], bc[0]), (bv[1], bc[1])),
                             mrg((bv[2], bc[2]), (bv[3], bc[3])))
                m = splat_min(v)
                gi = jnp.where(v == m, cid * np.float32(L) + lane_f,
                               np.float32(2 * N))
                n_f = splat_min(gi)
                n_f = jnp.where(m > OPENTH, np.float32(0.0), n_f)
                n_i = n_f.astype(jnp.int32)

                ny = n_i // W
                nx = n_i - ny * W
                my = ny + off_y
                mx = nx + off_x
                inb = is_nb & (my >= 0) & (my < W) & (mx >= 0) & (mx < W)
                mi = jnp.where(inb, my * W + mx, n_i)

                gm = plsc.load_gather(g_v, [mi])
                hm = plsc.load_gather(h_v, [mi])
                fm = plsc.load_gather(fkey_v, [mi])

                hn8 = hm.at[eights].get(mode="promise_in_bounds")
                gval = (gm.at[eights].get(mode="promise_in_bounds")
                        + jnp.where(hn8 > 0.0, np.float32(1.0),
                                    np.float32(0.0)))

                openm = fm < OPENTH
                never = fm == INF
                cond = inb & (hm > 0.0) & (
                    (openm & (gm > gval))
                    | (jnp.logical_not(openm) & never))

                newf = HALF * gval + HALF * jnp.abs(hm)
                close8 = lane8 & (n_f != goal_f)

                plsc.store_scatter(g_v, [mi], gval, mask=cond)
                plsc.store_scatter(parents_v, [mi], n_i, mask=cond)
                plsc.store_scatter(
                    fkey_v, [mi],
                    jnp.where(close8, CLOSEDV, newf),
                    mask=cond | close8)
                return t + 1, solved | (n_f == goal_f)

            _, solved_b = lax.while_loop(
                step_cond, step, (jnp.int32(0), jnp.zeros((L,), jnp.bool_)))
            solved = jnp.where(solved_b, 1.0, 0.0)

            for c in range(NCH):
                sl = pl.ds(c * L, L)
                hist_v[sl] = jnp.where(fkey_v[sl] == CLOSEDV, 1.0, 0.0)
            plsc.store_scatter(hist_v, [goal_i], solved, mask=lane0)

            loc0 = plsc.load_gather(parents_v, [goal_i])

            def bt_cond(carry):
                t, loc = carry
                return (t < T) & jnp.logical_not(jnp.any(loc == goal_i))

            def bt(carry):
                t, loc = carry
                plsc.store_scatter(path_v, [loc], ones_f, mask=lane0)
                return t + 1, plsc.load_gather(parents_v, [loc])

            lax.while_loop(bt_cond, bt, (jnp.int32(0), loc0))

            pltpu.sync_copy(hist_v, hist_out.at[wid])
            pltpu.sync_copy(path_v, path_out.at[wid])

    return k


def kernel(map_designs, start_maps, goal_maps):
    B, H, W = map_designs.shape
    N = H * W
    T = N // 2
    f32 = jnp.float32

    goal_idx = jnp.argmax(goal_maps.reshape(B, N), axis=1).astype(jnp.int32)
    start_idx = jnp.argmax(start_maps.reshape(B, N), axis=1).astype(jnp.int32)

    gy = (goal_idx // W).astype(f32)
    gx = (goal_idx % W).astype(f32)
    dy = jnp.abs(jnp.arange(H, dtype=f32)[None, :, None] - gy[:, None, None])
    dx = jnp.abs(jnp.arange(W, dtype=f32)[None, None, :] - gx[:, None, None])
    dy = jnp.broadcast_to(dy, (B, H, W))
    dx = jnp.broadcast_to(dx, (B, H, W))
    h_cheb = (dy + dx) - jnp.minimum(dy, dx)
    euc = jnp.sqrt(dy ** 2 + dx ** 2)
    h = (h_cheb + 0.001 * euc + map_designs).reshape(B, N).astype(f32)
    h_pack = jnp.where(map_designs.reshape(B, N) == 1.0, h, -h)

    b_ar = jnp.arange(B)
    fkey0 = jnp.full((B, N), jnp.inf, f32)
    fkey0 = fkey0.at[b_ar, start_idx].set(0.5 * h[b_ar, start_idx])
    parents0 = jnp.broadcast_to(goal_idx[:, None], (B, N))
    path0 = goal_maps.reshape(B, N).astype(f32)
    zeros_n = jnp.zeros((N,), f32)
    goal_splat = jnp.broadcast_to(goal_idx[:, None].astype(f32), (B, L))

    hist, path = _astar_sc_kernel(B, N, W, T)(
        h_pack, fkey0, parents0, path0, zeros_n, goal_splat)
    return hist.reshape(B, H, W), path.reshape(B, H, W)


# speedup vs baseline: 108.6504x; 1.0289x over previous
import functools

import jax
import jax.numpy as jnp
import numpy as np
from jax import lax
from jax.experimental import pallas as pl
from jax.experimental.pallas import tpu as pltpu
from jax.experimental.pallas import tpu_sc as plsc

L = 16
NC = 1
NS = 16
OPENTH = np.float32(1e36)
CLOSEDV = np.float32(1e37)
INF = np.float32(np.inf)
HALF = np.float32(0.5)


def _astar_sc_kernel(B, N, W, T):
    NCH = N // L
    mesh = plsc.VectorSubcoreMesh(
        core_axis_name="c", subcore_axis_name="s", num_cores=NC, num_subcores=NS
    )

    @functools.partial(
        pl.kernel,
        mesh=mesh,
        compiler_params=pltpu.CompilerParams(needs_layout_passes=False),
        out_type=[
            jax.ShapeDtypeStruct((B, N), jnp.float32),
            jax.ShapeDtypeStruct((B, N), jnp.float32),
        ],
        scratch_types=[
            pltpu.VMEM((N,), jnp.float32),
            pltpu.VMEM((N,), jnp.float32),
            pltpu.VMEM((N,), jnp.float32),
            pltpu.VMEM((N,), jnp.float32),
            pltpu.VMEM((N,), jnp.int32),
            pltpu.VMEM((N,), jnp.float32),
            pltpu.VMEM((L,), jnp.float32),
        ],
    )
    def k(h_hbm, fkey0_hbm, parents0_hbm, path0_hbm, zeros_hbm, goal_hbm,
          hist_out, path_out,
          fkey_v, g_v, h_v, hist_v, parents_v, path_v, goal_v):
        wid = lax.axis_index("s") * NC + lax.axis_index("c")

        @pl.when(wid < B)
        def _():
            pltpu.sync_copy(h_hbm.at[wid], h_v)
            pltpu.sync_copy(fkey0_hbm.at[wid], fkey_v)
            pltpu.sync_copy(parents0_hbm.at[wid], parents_v)
            pltpu.sync_copy(path0_hbm.at[wid], path_v)
            pltpu.sync_copy(zeros_hbm, g_v)
            pltpu.sync_copy(goal_hbm.at[wid], goal_v)

            lane = lax.iota(jnp.int32, L)
            lane_f = lane.astype(jnp.float32)
            goal_f = goal_v[...]
            goal_i = goal_f.astype(jnp.int32)
            t = lane + (lane >= 4).astype(jnp.int32)
            is_nb = lane < 8
            off_y = jnp.where(is_nb, t // 3 - 1, 0)
            off_x = jnp.where(is_nb, t % 3 - 1, 0)
            ones_f = jnp.full((L,), 1.0, jnp.float32)
            inf_v = jnp.full((L,), INF, jnp.float32)
            lane8 = lane == 8
            lane0 = lane == 0
            eights = jnp.full((L,), 8, jnp.int32)

            def splat_min(x):
                for kk in (1, 2, 4, 8):
                    x = jnp.minimum(
                        x, x.at[lane ^ kk].get(mode="promise_in_bounds"))
                return x

            def step_cond(carry):
                t, solved = carry
                return (t < T) & jnp.logical_not(jnp.any(solved))

            def step(carry):
                t, solved = carry
                bv = [inf_v, inf_v, inf_v, inf_v]
                bc = [lane_f * 0.0] * 4
                for c in range(NCH):
                    v = fkey_v[pl.ds(c * L, L)]
                    a = c & 3
                    lt = v < bv[a]
                    bv[a] = jnp.where(lt, v, bv[a])
                    bc[a] = jnp.where(lt, np.float32(c), bc[a])

                def mrg(p, q):
                    v0, c0 = p
                    v1, c1 = q
                    take1 = (v1 < v0) | ((v1 == v0) & (c1 < c0))
                    return (jnp.where(take1, v1, v0),
                            jnp.where(take1, c1, c0))

                v, cid = mrg(mrg((bv[0], bc[0]), (bv[1], bc[1])),
                             mrg((bv[2], bc[2]), (bv[3], bc[3])))
                m = splat_min(v)
                gi = jnp.where(v == m, cid * np.float32(L) + lane_f,
                               np.float32(2 * N))
                n_f = splat_min(gi)
                n_f = jnp.where(m > OPENTH, np.float32(0.0), n_f)
                n_i = n_f.astype(jnp.int32)

                ny = n_i // W
                nx = n_i - ny * W
                my = ny + off_y
                mx = nx + off_x
                inb = is_nb & (my >= 0) & (my < W) & (mx >= 0) & (mx < W)
                mi = jnp.where(inb, my * W + mx, n_i)

                gm = plsc.load_gather(g_v, [mi])
                hm = plsc.load_gather(h_v, [mi])
                fm = plsc.load_gather(fkey_v, [mi])

                hn8 = hm.at[eights].get(mode="promise_in_bounds")
                gval = (gm.at[eights].get(mode="promise_in_bounds")
                        + jnp.where(hn8 > 0.0, np.float32(1.0),
                                    np.float32(0.0)))

                openm = fm < OPENTH
                never = fm == INF
                cond = inb & (hm > 0.0) & (
                    (openm & (gm > gval))
                    | (jnp.logical_not(openm) & never))

                newf = HALF * gval + HALF * jnp.abs(hm)
                close8 = lane8 & (n_f != goal_f)

                plsc.store_scatter(g_v, [mi], gval, mask=cond)
                plsc.store_scatter(parents_v, [mi], n_i, mask=cond)
                plsc.store_scatter(
                    fkey_v, [mi],
                    jnp.where(close8, CLOSEDV, newf),
                    mask=cond | close8)
                return t + 1, solved | (n_f == goal_f)

            _, solved_b = lax.while_loop(
                step_cond, step, (jnp.int32(0), jnp.zeros((L,), jnp.bool_)))
            solved = jnp.where(solved_b, 1.0, 0.0)

            for c in range(NCH):
                sl = pl.ds(c * L, L)
                hist_v[sl] = jnp.where(fkey_v[sl] == CLOSEDV, 1.0, 0.0)
            plsc.store_scatter(hist_v, [goal_i], solved, mask=lane0)

            loc0 = plsc.load_gather(parents_v, [goal_i])

            def bt_cond(carry):
                t, loc = carry
                return (t < T) & jnp.logical_not(jnp.any(loc == goal_i))

            def bt(carry):
                t, loc = carry
                plsc.store_scatter(path_v, [loc], ones_f, mask=lane0)
                return t + 1, plsc.load_gather(parents_v, [loc])

            lax.while_loop(bt_cond, bt, (jnp.int32(0), loc0))

            pltpu.sync_copy(hist_v, hist_out.at[wid])
            pltpu.sync_copy(path_v, path_out.at[wid])

    return k


def kernel(map_designs, start_maps, goal_maps):
    B, H, W = map_designs.shape
    N = H * W
    T = N // 2
    f32 = jnp.float32

    goal_idx = jnp.argmax(goal_maps.reshape(B, N), axis=1).astype(jnp.int32)
    start_idx = jnp.argmax(start_maps.reshape(B, N), axis=1).astype(jnp.int32)

    gy = (goal_idx // W).astype(f32)
    gx = (goal_idx % W).astype(f32)
    dy = jnp.abs(jnp.arange(H, dtype=f32)[None, :, None] - gy[:, None, None])
    dx = jnp.abs(jnp.arange(W, dtype=f32)[None, None, :] - gx[:, None, None])
    dy = jnp.broadcast_to(dy, (B, H, W))
    dx = jnp.broadcast_to(dx, (B, H, W))
    h_cheb = (dy + dx) - jnp.minimum(dy, dx)
    euc = jnp.sqrt(dy ** 2 + dx ** 2)
    h = (h_cheb + 0.001 * euc + map_designs).reshape(B, N).astype(f32)
    h_pack = jnp.where(map_designs.reshape(B, N) == 1.0, h, -h)

    b_ar = jnp.arange(B)
    fkey0 = jnp.full((B, N), jnp.inf, f32)
    fkey0 = fkey0.at[b_ar, start_idx].set(0.5 * h[b_ar, start_idx])
    parents0 = jnp.broadcast_to(goal_idx[:, None], (B, N))
    path0 = goal_maps.reshape(B, N).astype(f32)
    zeros_n = jnp.zeros((N,), f32)
    goal_splat = jnp.broadcast_to(goal_idx[:, None].astype(f32), (B, L))

    hist, path = _astar_sc_kernel(B, N, W, T)(
        h_pack, fkey0, parents0, path0, zeros_n, goal_splat)
    return hist.reshape(B, H, W), path.reshape(B, H, W)


# speedup vs baseline: 110.5261x; 1.0173x over previous
import functools

import jax
import jax.numpy as jnp
import numpy as np
from jax import lax
from jax.experimental import pallas as pl
from jax.experimental.pallas import tpu as pltpu
from jax.experimental.pallas import tpu_sc as plsc

L = 16
NC = 1
NS = 16
OPENTH = np.float32(1e36)
CLOSEDV = np.float32(1e37)
INF = np.float32(np.inf)
HALF = np.float32(0.5)


def _astar_sc_kernel(B, N, W, T):
    NCH = N // L
    mesh = plsc.VectorSubcoreMesh(
        core_axis_name="c", subcore_axis_name="s", num_cores=NC, num_subcores=NS
    )

    @functools.partial(
        pl.kernel,
        mesh=mesh,
        compiler_params=pltpu.CompilerParams(needs_layout_passes=False),
        out_type=[
            jax.ShapeDtypeStruct((B, N), jnp.float32),
            jax.ShapeDtypeStruct((B, N), jnp.float32),
        ],
        scratch_types=[
            pltpu.VMEM((N,), jnp.float32),
            pltpu.VMEM((N,), jnp.float32),
            pltpu.VMEM((N,), jnp.float32),
            pltpu.VMEM((N,), jnp.float32),
            pltpu.VMEM((N,), jnp.int32),
            pltpu.VMEM((N,), jnp.float32),
            pltpu.VMEM((L,), jnp.float32),
            pltpu.SemaphoreType.DMA,
        ],
    )
    def k(h_hbm, fkey0_hbm, parents0_hbm, path0_hbm, zeros_hbm, goal_hbm,
          hist_out, path_out,
          fkey_v, g_v, h_v, hist_v, parents_v, path_v, goal_v, sem):
        wid = lax.axis_index("s") * NC + lax.axis_index("c")

        @pl.when(wid < B)
        def _():
            cps = [
                pltpu.async_copy(h_hbm.at[wid], h_v, sem),
                pltpu.async_copy(fkey0_hbm.at[wid], fkey_v, sem),
                pltpu.async_copy(parents0_hbm.at[wid], parents_v, sem),
                pltpu.async_copy(path0_hbm.at[wid], path_v, sem),
                pltpu.async_copy(zeros_hbm, g_v, sem),
                pltpu.async_copy(goal_hbm.at[wid], goal_v, sem),
            ]
            for cp in cps:
                cp.wait()

            lane = lax.iota(jnp.int32, L)
            lane_f = lane.astype(jnp.float32)
            goal_f = goal_v[...]
            goal_i = goal_f.astype(jnp.int32)
            t = lane + (lane >= 4).astype(jnp.int32)
            is_nb = lane < 8
            off_y = jnp.where(is_nb, t // 3 - 1, 0)
            off_x = jnp.where(is_nb, t % 3 - 1, 0)
            ones_f = jnp.full((L,), 1.0, jnp.float32)
            inf_v = jnp.full((L,), INF, jnp.float32)
            lane8 = lane == 8
            lane0 = lane == 0
            eights = jnp.full((L,), 8, jnp.int32)

            def splat_min(x):
                for kk in (1, 2, 4, 8):
                    x = jnp.minimum(
                        x, x.at[lane ^ kk].get(mode="promise_in_bounds"))
                return x

            def step_cond(carry):
                t, solved = carry
                return (t < T) & jnp.logical_not(jnp.any(solved))

            def step(carry):
                t, solved = carry
                NS_ = 8
                bv = [inf_v] * NS_
                bc = [lane_f * 0.0] * NS_
                for c in range(NCH):
                    v = fkey_v[pl.ds(c * L, L)]
                    a = c & (NS_ - 1)
                    lt = v < bv[a]
                    bv[a] = jnp.where(lt, v, bv[a])
                    bc[a] = jnp.where(lt, np.float32(c), bc[a])

                def mrg(p, q):
                    v0, c0 = p
                    v1, c1 = q
                    take1 = (v1 < v0) | ((v1 == v0) & (c1 < c0))
                    return (jnp.where(take1, v1, v0),
                            jnp.where(take1, c1, c0))

                ps = [(bv[a], bc[a]) for a in range(NS_)]
                while len(ps) > 1:
                    ps = [mrg(ps[i], ps[i + 1])
                          for i in range(0, len(ps), 2)]
                v, cid = ps[0]
                m = splat_min(v)
                gi = jnp.where(v == m, cid * np.float32(L) + lane_f,
                               np.float32(2 * N))
                n_f = splat_min(gi)
                n_f = jnp.where(m > OPENTH, np.float32(0.0), n_f)
                n_i = n_f.astype(jnp.int32)

                ny = n_i // W
                nx = n_i - ny * W
                my = ny + off_y
                mx = nx + off_x
                inb = is_nb & (my >= 0) & (my < W) & (mx >= 0) & (mx < W)
                mi = jnp.where(inb, my * W + mx, n_i)

                gm = plsc.load_gather(g_v, [mi])
                hm = plsc.load_gather(h_v, [mi])
                fm = plsc.load_gather(fkey_v, [mi])

                hn8 = hm.at[eights].get(mode="promise_in_bounds")
                gval = (gm.at[eights].get(mode="promise_in_bounds")
                        + jnp.where(hn8 > 0.0, np.float32(1.0),
                                    np.float32(0.0)))

                openm = fm < OPENTH
                never = fm == INF
                cond = inb & (hm > 0.0) & (
                    (openm & (gm > gval))
                    | (jnp.logical_not(openm) & never))

                newf = HALF * gval + HALF * jnp.abs(hm)
                close8 = lane8 & (n_f != goal_f)

                plsc.store_scatter(g_v, [mi], gval, mask=cond)
                plsc.store_scatter(parents_v, [mi], n_i, mask=cond)
                plsc.store_scatter(
                    fkey_v, [mi],
                    jnp.where(close8, CLOSEDV, newf),
                    mask=cond | close8)
                return t + 1, solved | (n_f == goal_f)

            _, solved_b = lax.while_loop(
                step_cond, step, (jnp.int32(0), jnp.zeros((L,), jnp.bool_)))
            solved = jnp.where(solved_b, 1.0, 0.0)

            for c in range(NCH):
                sl = pl.ds(c * L, L)
                hist_v[sl] = jnp.where(fkey_v[sl] == CLOSEDV, 1.0, 0.0)
            plsc.store_scatter(hist_v, [goal_i], solved, mask=lane0)

            loc0 = plsc.load_gather(parents_v, [goal_i])

            def bt_cond(carry):
                t, loc = carry
                return (t < T) & jnp.logical_not(jnp.any(loc == goal_i))

            def bt(carry):
                t, loc = carry
                plsc.store_scatter(path_v, [loc], ones_f, mask=lane0)
                return t + 1, plsc.load_gather(parents_v, [loc])

            lax.while_loop(bt_cond, bt, (jnp.int32(0), loc0))

            pltpu.sync_copy(hist_v, hist_out.at[wid])
            pltpu.sync_copy(path_v, path_out.at[wid])

    return k


def kernel(map_designs, start_maps, goal_maps):
    B, H, W = map_designs.shape
    N = H * W
    T = N // 2
    f32 = jnp.float32

    goal_idx = jnp.argmax(goal_maps.reshape(B, N), axis=1).astype(jnp.int32)
    start_idx = jnp.argmax(start_maps.reshape(B, N), axis=1).astype(jnp.int32)

    gy = (goal_idx // W).astype(f32)
    gx = (goal_idx % W).astype(f32)
    dy = jnp.abs(jnp.arange(H, dtype=f32)[None, :, None] - gy[:, None, None])
    dx = jnp.abs(jnp.arange(W, dtype=f32)[None, None, :] - gx[:, None, None])
    dy = jnp.broadcast_to(dy, (B, H, W))
    dx = jnp.broadcast_to(dx, (B, H, W))
    h_cheb = (dy + dx) - jnp.minimum(dy, dx)
    euc = jnp.sqrt(dy ** 2 + dx ** 2)
    h = (h_cheb + 0.001 * euc + map_designs).reshape(B, N).astype(f32)
    h_pack = jnp.where(map_designs.reshape(B, N) == 1.0, h, -h)

    b_ar = jnp.arange(B)
    fkey0 = jnp.full((B, N), jnp.inf, f32)
    fkey0 = fkey0.at[b_ar, start_idx].set(0.5 * h[b_ar, start_idx])
    parents0 = jnp.broadcast_to(goal_idx[:, None], (B, N))
    path0 = goal_maps.reshape(B, N).astype(f32)
    zeros_n = jnp.zeros((N,), f32)
    goal_splat = jnp.broadcast_to(goal_idx[:, None].astype(f32), (B, L))

    hist, path = _astar_sc_kernel(B, N, W, T)(
        h_pack, fkey0, parents0, path0, zeros_n, goal_splat)
    return hist.reshape(B, H, W), path.reshape(B, H, W)


# speedup vs baseline: 112.2512x; 1.0156x over previous
import functools

import jax
import jax.numpy as jnp
import numpy as np
from jax import lax
from jax.experimental import pallas as pl
from jax.experimental.pallas import tpu as pltpu
from jax.experimental.pallas import tpu_sc as plsc

L = 16
NC = 1
NS = 16
OPENTH = np.float32(1e36)
CLOSEDV = np.float32(1e37)
INF = np.float32(np.inf)
HALF = np.float32(0.5)


def _astar_sc_kernel(B, N, W, T):
    NCH = N // L
    mesh = plsc.VectorSubcoreMesh(
        core_axis_name="c", subcore_axis_name="s", num_cores=NC, num_subcores=NS
    )

    @functools.partial(
        pl.kernel,
        mesh=mesh,
        compiler_params=pltpu.CompilerParams(needs_layout_passes=False),
        out_type=[
            jax.ShapeDtypeStruct((B, N), jnp.float32),
            jax.ShapeDtypeStruct((B, N), jnp.float32),
        ],
        scratch_types=[
            pltpu.VMEM((N,), jnp.float32),
            pltpu.VMEM((N,), jnp.float32),
            pltpu.VMEM((N,), jnp.float32),
            pltpu.VMEM((N,), jnp.float32),
            pltpu.VMEM((N,), jnp.int32),
            pltpu.VMEM((N,), jnp.float32),
            pltpu.VMEM((L,), jnp.float32),
            pltpu.SemaphoreType.DMA,
        ],
    )
    def k(h_hbm, fkey0_hbm, parents0_hbm, path0_hbm, zeros_hbm, goal_hbm,
          hist_out, path_out,
          fkey_v, g_v, h_v, hist_v, parents_v, path_v, goal_v, sem):
        wid = lax.axis_index("s") * NC + lax.axis_index("c")

        @pl.when(wid < B)
        def _():
            cps = [
                pltpu.async_copy(h_hbm.at[wid], h_v, sem),
                pltpu.async_copy(fkey0_hbm.at[wid], fkey_v, sem),
                pltpu.async_copy(parents0_hbm.at[wid], parents_v, sem),
                pltpu.async_copy(path0_hbm.at[wid], path_v, sem),
                pltpu.async_copy(zeros_hbm, g_v, sem),
                pltpu.async_copy(goal_hbm.at[wid], goal_v, sem),
            ]
            for cp in cps:
                cp.wait()

            lane = lax.iota(jnp.int32, L)
            lane_f = lane.astype(jnp.float32)
            goal_f = goal_v[...]
            goal_i = goal_f.astype(jnp.int32)
            t = lane + (lane >= 4).astype(jnp.int32)
            is_nb = lane < 8
            off_y = jnp.where(is_nb, t // 3 - 1, 0)
            off_x = jnp.where(is_nb, t % 3 - 1, 0)
            ones_f = jnp.full((L,), 1.0, jnp.float32)
            inf_v = jnp.full((L,), INF, jnp.float32)
            lane8 = lane == 8
            lane0 = lane == 0
            eights = jnp.full((L,), 8, jnp.int32)

            def splat_min(x):
                for kk in (1, 2, 4, 8):
                    x = jnp.minimum(
                        x, x.at[lane ^ kk].get(mode="promise_in_bounds"))
                return x

            def step_cond(carry):
                t, solved = carry
                return (t < T) & jnp.logical_not(jnp.any(solved))

            def step(carry):
                t, solved = carry
                NS_ = 8
                bv = [inf_v] * NS_
                bc = [lane_f * 0.0] * NS_
                for c in range(NCH):
                    v = fkey_v[pl.ds(c * L, L)]
                    a = c & (NS_ - 1)
                    lt = v < bv[a]
                    bv[a] = jnp.where(lt, v, bv[a])
                    bc[a] = jnp.where(lt, np.float32(c), bc[a])

                def mrg(p, q):
                    v0, c0 = p
                    v1, c1 = q
                    take1 = (v1 < v0) | ((v1 == v0) & (c1 < c0))
                    return (jnp.where(take1, v1, v0),
                            jnp.where(take1, c1, c0))

                ps = [(bv[a], bc[a]) for a in range(NS_)]
                while len(ps) > 1:
                    ps = [mrg(ps[i], ps[i + 1])
                          for i in range(0, len(ps), 2)]
                v, cid = ps[0]
                m = splat_min(v)
                gi = jnp.where(v == m, cid * np.float32(L) + lane_f,
                               np.float32(2 * N))
                n_f = splat_min(gi)
                n_f = jnp.where(m > OPENTH, np.float32(0.0), n_f)
                n_i = n_f.astype(jnp.int32)

                wsh = W.bit_length() - 1
                ny = n_i >> wsh
                nx = n_i & (W - 1)
                my = ny + off_y
                mx = nx + off_x
                inb = is_nb & (my >= 0) & (my < W) & (mx >= 0) & (mx < W)
                mi = jnp.where(inb, (my << wsh) + mx, n_i)

                gm = plsc.load_gather(g_v, [mi])
                hm = plsc.load_gather(h_v, [mi])
                fm = plsc.load_gather(fkey_v, [mi])

                hn8 = hm.at[eights].get(mode="promise_in_bounds")
                gval = (gm.at[eights].get(mode="promise_in_bounds")
                        + jnp.where(hn8 > 0.0, np.float32(1.0),
                                    np.float32(0.0)))

                openm = fm < OPENTH
                never = fm == INF
                cond = inb & (hm > 0.0) & (
                    (openm & (gm > gval))
                    | (jnp.logical_not(openm) & never))

                newf = HALF * gval + HALF * jnp.abs(hm)
                close8 = lane8 & (n_f != goal_f)

                plsc.store_scatter(g_v, [mi], gval, mask=cond)
                plsc.store_scatter(parents_v, [mi], n_i, mask=cond)
                plsc.store_scatter(
                    fkey_v, [mi],
                    jnp.where(close8, CLOSEDV, newf),
                    mask=cond | close8)
                return t + 1, solved | (n_f == goal_f)

            _, solved_b = lax.while_loop(
                step_cond, step, (jnp.int32(0), jnp.zeros((L,), jnp.bool_)))
            solved = jnp.where(solved_b, 1.0, 0.0)

            for c in range(NCH):
                sl = pl.ds(c * L, L)
                hist_v[sl] = jnp.where(fkey_v[sl] == CLOSEDV, 1.0, 0.0)
            plsc.store_scatter(hist_v, [goal_i], solved, mask=lane0)

            loc0 = plsc.load_gather(parents_v, [goal_i])

            def bt_cond(carry):
                t, loc = carry
                return (t < T) & jnp.logical_not(jnp.any(loc == goal_i))

            def bt(carry):
                t, loc = carry
                plsc.store_scatter(path_v, [loc], ones_f, mask=lane0)
                return t + 1, plsc.load_gather(parents_v, [loc])

            lax.while_loop(bt_cond, bt, (jnp.int32(0), loc0))

            pltpu.sync_copy(hist_v, hist_out.at[wid])
            pltpu.sync_copy(path_v, path_out.at[wid])

    return k


def kernel(map_designs, start_maps, goal_maps):
    B, H, W = map_designs.shape
    N = H * W
    T = N // 2
    f32 = jnp.float32

    goal_idx = jnp.argmax(goal_maps.reshape(B, N), axis=1).astype(jnp.int32)
    start_idx = jnp.argmax(start_maps.reshape(B, N), axis=1).astype(jnp.int32)

    gy = (goal_idx // W).astype(f32)
    gx = (goal_idx % W).astype(f32)
    dy = jnp.abs(jnp.arange(H, dtype=f32)[None, :, None] - gy[:, None, None])
    dx = jnp.abs(jnp.arange(W, dtype=f32)[None, None, :] - gx[:, None, None])
    dy = jnp.broadcast_to(dy, (B, H, W))
    dx = jnp.broadcast_to(dx, (B, H, W))
    h_cheb = (dy + dx) - jnp.minimum(dy, dx)
    euc = jnp.sqrt(dy ** 2 + dx ** 2)
    h = (h_cheb + 0.001 * euc + map_designs).reshape(B, N).astype(f32)
    h_pack = jnp.where(map_designs.reshape(B, N) == 1.0, h, -h)

    b_ar = jnp.arange(B)
    fkey0 = jnp.full((B, N), jnp.inf, f32)
    fkey0 = fkey0.at[b_ar, start_idx].set(0.5 * h[b_ar, start_idx])
    parents0 = jnp.broadcast_to(goal_idx[:, None], (B, N))
    path0 = goal_maps.reshape(B, N).astype(f32)
    zeros_n = jnp.zeros((N,), f32)
    goal_splat = jnp.broadcast_to(goal_idx[:, None].astype(f32), (B, L))

    hist, path = _astar_sc_kernel(B, N, W, T)(
        h_pack, fkey0, parents0, path0, zeros_n, goal_splat)
    return hist.reshape(B, H, W), path.reshape(B, H, W)
